# Initial kernel scaffold; baseline (speedup 1.0000x reference)
import jax
import jax.numpy as jnp
from jax.experimental import pallas as pl


def kernel(input_ids, vocab_W, pos_W):
    raise NotImplementedError("write your pallas kernel here")


# speedup vs baseline: 1.3538x; 1.3538x over previous
import functools

import jax
import jax.numpy as jnp
from jax import lax
from jax.experimental import pallas as pl
from jax.experimental.pallas import tpu as pltpu
from jax.experimental.pallas import tpu_sc as plsc

VOCAB = 100000
DIM = 768
SEQ = 2048
BATCH = 4

NC = 2
NS = 16
LANES = 16
NW = NC * NS
ROWS = BATCH * SEQ
ROWS_PER_W = ROWS // NW
CHUNK = 64
NCHUNK = ROWS_PER_W // CHUNK
DSLICES = DIM // LANES


def _body(ids_hbm, vocab_hbm, pos_hbm, out_hbm, idx_v, rows_v, pos_v,
          gsem, psem):
    c_i = lax.axis_index("c")
    s_i = lax.axis_index("s")
    wid = s_i * NC + c_i
    base = wid * ROWS_PER_W
    pos_base = lax.rem(wid, SEQ // ROWS_PER_W) * ROWS_PER_W

    pltpu.sync_copy(ids_hbm.at[wid], idx_v)

    for c in range(NCHUNK):
        g = pltpu.async_copy(vocab_hbm.at[idx_v.at[c]], rows_v, gsem)
        p = pltpu.async_copy(
            pos_hbm.at[pl.ds(pos_base + c * CHUNK, CHUNK)], pos_v, psem)
        g.wait()
        p.wait()

        def row_body(r, _):
            for d in range(DSLICES):
                sl = pl.ds(d * LANES, LANES)
                rows_v[r, sl] = rows_v[r, sl] + pos_v[r, sl]
            return 0

        lax.fori_loop(0, CHUNK, row_body, 0)
        pltpu.sync_copy(rows_v, out_hbm.at[pl.ds(base + c * CHUNK, CHUNK)])


@jax.jit
def kernel(input_ids, vocab_W, pos_W):
    ids3 = input_ids.reshape(NW, NCHUNK, CHUNK).astype(jnp.int32)
    mesh = plsc.VectorSubcoreMesh(core_axis_name="c", subcore_axis_name="s")
    run = pl.kernel(
        _body,
        out_type=jax.ShapeDtypeStruct((ROWS, DIM), jnp.float32),
        mesh=mesh,
        scratch_types=[
            pltpu.VMEM((NCHUNK, CHUNK), jnp.int32),
            pltpu.VMEM((CHUNK, DIM), jnp.float32),
            pltpu.VMEM((CHUNK, DIM), jnp.float32),
            pltpu.SemaphoreType.DMA,
            pltpu.SemaphoreType.DMA,
        ],
    )
    out = run(ids3, vocab_W, pos_W)
    return out.reshape(BATCH, SEQ, DIM)


# speedup vs baseline: 1.4867x; 1.0982x over previous
import jax
import jax.numpy as jnp
from jax import lax
from jax.experimental import pallas as pl
from jax.experimental.pallas import tpu as pltpu
from jax.experimental.pallas import tpu_sc as plsc

VOCAB = 100000
DIM = 768
SEQ = 2048
BATCH = 4

NC = 2
NS = 16
LANES = 16
NW = NC * NS
ROWS = BATCH * SEQ
ROWS_PER_W = ROWS // NW
CHUNK = 16
NCHUNK = ROWS_PER_W // CHUNK
DSLICES = DIM // LANES
NRB = 4
NPB = 3


def _body(ids_hbm, vocab_hbm, pos_hbm, out_hbm, idx_v,
          r0, r1, r2, r3, p0, p1, p2,
          gs0, gs1, gs2, gs3, ps0, ps1, ps2, os0, os1, os2, os3):
    rows = [r0, r1, r2, r3]
    poss = [p0, p1, p2]
    gsem = [gs0, gs1, gs2, gs3]
    psem = [ps0, ps1, ps2]
    osem = [os0, os1, os2, os3]

    c_i = lax.axis_index("c")
    s_i = lax.axis_index("s")
    wid = s_i * NC + c_i
    base = wid * ROWS_PER_W
    pos_base = lax.rem(wid, SEQ // ROWS_PER_W) * ROWS_PER_W

    pltpu.sync_copy(ids_hbm.at[wid], idx_v)

    g, p, o = {}, {}, {}

    def start(c):
        rb, pb = c % NRB, c % NPB
        g[c] = pltpu.async_copy(vocab_hbm.at[idx_v.at[c]], rows[rb], gsem[rb])
        p[c] = pltpu.async_copy(
            pos_hbm.at[pl.ds(pos_base + c * CHUNK, CHUNK)], poss[pb],
            psem[pb])

    start(0)
    start(1)
    for c in range(NCHUNK):
        if c + 2 < NCHUNK:
            if c >= 2:
                o[c - 2].wait()
            start(c + 2)
        g[c].wait()
        p[c].wait()
        rb, pb = c % NRB, c % NPB

        def row_body(r, _, rb=rb, pb=pb):
            for d in range(DSLICES):
                sl = pl.ds(d * LANES, LANES)
                plsc.addupdate(rows[rb].at[r, sl], poss[pb][r, sl])
            return 0

        lax.fori_loop(0, CHUNK, row_body, 0)
        o[c] = pltpu.async_copy(
            rows[rb], out_hbm.at[pl.ds(base + c * CHUNK, CHUNK)], osem[rb])
    for c in range(max(0, NCHUNK - 4), NCHUNK):
        o[c].wait()


@jax.jit
def kernel(input_ids, vocab_W, pos_W):
    ids3 = input_ids.reshape(NW, NCHUNK, CHUNK).astype(jnp.int32)
    mesh = plsc.VectorSubcoreMesh(core_axis_name="c", subcore_axis_name="s")
    run = pl.kernel(
        _body,
        out_type=jax.ShapeDtypeStruct((ROWS, DIM), jnp.float32),
        mesh=mesh,
        scratch_types=(
            [pltpu.VMEM((NCHUNK, CHUNK), jnp.int32)]
            + [pltpu.VMEM((CHUNK, DIM), jnp.float32) for _ in range(NRB)]
            + [pltpu.VMEM((CHUNK, DIM), jnp.float32) for _ in range(NPB)]
            + [pltpu.SemaphoreType.DMA for _ in range(NRB + NPB + NRB)]
        ),
    )
    out = run(ids3, vocab_W, pos_W)
    return out.reshape(BATCH, SEQ, DIM)
